# baseline (speedup 1.0000x reference)
import functools

import jax
import jax.numpy as jnp
from jax import lax
from jax.experimental import pallas as pl
from jax.experimental.pallas import tpu as pltpu
from jax.experimental.pallas import tpu_sc as plsc

N = 2048
D = 1024
K = 8192

_info = plsc.get_sparse_core_info()
_NC = _info.num_cores
_NS = _info.num_subcores
_NW = _NC * _NS
_BPW = N // _NW


def _sc_gather(codebook, target):
  mesh = plsc.VectorSubcoreMesh(core_axis_name="c", subcore_axis_name="s")

  @functools.partial(
      pl.kernel,
      mesh=mesh,
      out_type=jax.ShapeDtypeStruct((N, D), jnp.float32),
      scratch_types=[
          pltpu.VMEM((_BPW,), jnp.int32),
          pltpu.VMEM((_BPW, D), jnp.float32),
          pltpu.SemaphoreType.DMA,
      ],
  )
  def k(table_hbm, idx_hbm, out_hbm, idx_v, rows_v, sem):
    wid = lax.axis_index("s") * _NC + lax.axis_index("c")
    base = wid * _BPW
    pltpu.sync_copy(idx_hbm.at[pl.ds(base, _BPW)], idx_v)
    pltpu.async_copy(table_hbm.at[idx_v], rows_v, sem).wait()
    pltpu.sync_copy(rows_v, out_hbm.at[pl.ds(base, _BPW)])

  return k(codebook, target)


_BE = 256
_NB = N // _BE
_DP = 1152


def _loss_body(pred_ref, e_ref, mask_ref, out_ref, a_ref, b_ref):
  j = pl.program_id(0)
  nj = pl.num_programs(0)

  @pl.when(j == 0)
  def _():
    out_ref[0, 0] = 0.0

  @pl.when(j < _NB)
  def _():
    base = j * _BE
    zpad = jnp.zeros((_BE, _DP - D - 4), jnp.bfloat16)

    p = pred_ref[...]
    m = mask_ref[pl.ds(base, _BE), :]
    p2 = jnp.sum(p * p, axis=1, keepdims=True) * m
    p2hi = p2.astype(jnp.bfloat16)
    p2lo = (p2 - p2hi.astype(jnp.float32)).astype(jnp.bfloat16)
    mb = m.astype(jnp.bfloat16)
    a_ref[pl.ds(base, _BE), :D] = (p * (-2.0 * m)).astype(jnp.bfloat16)
    a_ref[pl.ds(base, _BE), D:] = jnp.concatenate(
        [p2hi, p2lo, mb, mb, zpad], axis=1)

    e = e_ref[...]
    t2 = jnp.sum(e * e, axis=1, keepdims=True)
    t2hi = t2.astype(jnp.bfloat16)
    t2lo = (t2 - t2hi.astype(jnp.float32)).astype(jnp.bfloat16)
    ones = jnp.ones((_BE, 1), jnp.bfloat16)
    b_ref[pl.ds(base, _BE), :D] = e.astype(jnp.bfloat16)
    b_ref[pl.ds(base, _BE), D:] = jnp.concatenate(
        [ones, ones, t2hi, t2lo, zpad], axis=1)

  @pl.when(j >= _NB)
  def _():
    jj = j - _NB
    d2 = lax.dot_general(
        a_ref[...], b_ref[pl.ds(jj * _BE, _BE), :],
        (((1,), (1,)), ((), ())),
        preferred_element_type=jnp.float32)
    d2 = jnp.maximum(d2, 1e-30)
    part = jnp.sum(d2 * lax.rsqrt(d2))
    acc = out_ref[0, 0] + part

    @pl.when(j < nj - 1)
    def _():
      out_ref[0, 0] = acc

    @pl.when(j == nj - 1)
    def _():
      msum = jnp.sum(mask_ref[...])
      out_ref[0, 0] = acc / (msum * D)


def kernel(pred, target, codebook):
  emb = _sc_gather(codebook, target)
  maskf = (target != -1).astype(jnp.float32).reshape(N, 1)

  out = pl.pallas_call(
      _loss_body,
      grid=(2 * _NB,),
      in_specs=[
          pl.BlockSpec((_BE, D), lambda j: (jnp.minimum(j, _NB - 1), 0)),
          pl.BlockSpec((_BE, D), lambda j: (jnp.minimum(j, _NB - 1), 0)),
          pl.BlockSpec((N, 1), lambda j: (0, 0)),
      ],
      out_specs=pl.BlockSpec(memory_space=pltpu.SMEM),
      out_shape=jax.ShapeDtypeStruct((1, 1), jnp.float32),
      scratch_shapes=[
          pltpu.VMEM((N, _DP), jnp.bfloat16),
          pltpu.VMEM((N, _DP), jnp.bfloat16),
      ],
  )(pred, emb, maskf)
  return out[0, 0]


# speedup vs baseline: 1.0117x; 1.0117x over previous
import functools

import jax
import jax.numpy as jnp
from jax import lax
from jax.experimental import pallas as pl
from jax.experimental.pallas import tpu as pltpu
from jax.experimental.pallas import tpu_sc as plsc

N = 2048
D = 1024
K = 8192

_info = plsc.get_sparse_core_info()
_NC = _info.num_cores
_NS = _info.num_subcores
_NW = _NC * _NS
_BPW = N // _NW


def _sc_gather(codebook, target):
  mesh = plsc.VectorSubcoreMesh(core_axis_name="c", subcore_axis_name="s")

  @functools.partial(
      pl.kernel,
      mesh=mesh,
      out_type=jax.ShapeDtypeStruct((N, D), jnp.float32),
      scratch_types=[
          pltpu.VMEM((_BPW,), jnp.int32),
          pltpu.VMEM((_BPW, D), jnp.float32),
          pltpu.SemaphoreType.DMA,
      ],
  )
  def k(table_hbm, idx_hbm, out_hbm, idx_v, rows_v, sem):
    wid = lax.axis_index("s") * _NC + lax.axis_index("c")
    base = wid * _BPW
    pltpu.sync_copy(idx_hbm.at[pl.ds(base, _BPW)], idx_v)
    pltpu.async_copy(table_hbm.at[idx_v], rows_v, sem).wait()
    pltpu.sync_copy(rows_v, out_hbm.at[pl.ds(base, _BPW)])

  return k(codebook, target)


_BE = 256
_NB = N // _BE
_DP = 1152


def _build_a_body(pred_ref, mask_ref, a_ref, msum_ref):
  j = pl.program_id(0)
  p = pred_ref[...]
  m = mask_ref[...]
  p2 = jnp.sum(p * p, axis=1, keepdims=True) * m
  p2hi = p2.astype(jnp.bfloat16)
  p2lo = (p2 - p2hi.astype(jnp.float32)).astype(jnp.bfloat16)
  mb = m.astype(jnp.bfloat16)
  a_ref[:, :D] = (p * (-2.0 * m)).astype(jnp.bfloat16)
  a_ref[:, D:] = jnp.concatenate(
      [p2hi, p2lo, mb, mb, jnp.zeros((_BE, _DP - D - 4), jnp.bfloat16)],
      axis=1)

  @pl.when(j == 0)
  def _():
    msum_ref[0, 0] = 0.0
  msum_ref[0, 0] += jnp.sum(m)


def _loss_body(a_ref, e_ref, msum_ref, out_ref, b_ref):
  j = pl.program_id(0)
  nj = pl.num_programs(0)

  @pl.when(j == 0)
  def _():
    out_ref[0, 0] = 0.0

  @pl.when(j < _NB)
  def _():
    base = j * _BE
    e = e_ref[...]
    t2 = jnp.sum(e * e, axis=1, keepdims=True)
    t2hi = t2.astype(jnp.bfloat16)
    t2lo = (t2 - t2hi.astype(jnp.float32)).astype(jnp.bfloat16)
    ones = jnp.ones((_BE, 1), jnp.bfloat16)
    b_ref[pl.ds(base, _BE), :D] = e.astype(jnp.bfloat16)
    b_ref[pl.ds(base, _BE), D:] = jnp.concatenate(
        [ones, ones, t2hi, t2lo,
         jnp.zeros((_BE, _DP - D - 4), jnp.bfloat16)], axis=1)

  @pl.when(j >= 1)
  def _():
    jj = j - 1
    d2 = lax.dot_general(
        a_ref[...], b_ref[pl.ds(jj * _BE, _BE), :],
        (((1,), (1,)), ((), ())),
        preferred_element_type=jnp.float32)
    d2 = jnp.maximum(d2, 1e-30)
    part = jnp.sum(d2 * lax.rsqrt(d2))
    acc = out_ref[0, 0] + part

    @pl.when(j < nj - 1)
    def _():
      out_ref[0, 0] = acc

    @pl.when(j == nj - 1)
    def _():
      out_ref[0, 0] = acc / (msum_ref[0, 0] * D)


def kernel(pred, target, codebook):
  emb = _sc_gather(codebook, target)
  maskf = (target != -1).astype(jnp.float32).reshape(N, 1)

  a_mat, msum = pl.pallas_call(
      _build_a_body,
      grid=(_NB,),
      in_specs=[
          pl.BlockSpec((_BE, D), lambda j: (j, 0)),
          pl.BlockSpec((_BE, 1), lambda j: (j, 0)),
      ],
      out_specs=[
          pl.BlockSpec((_BE, _DP), lambda j: (j, 0)),
          pl.BlockSpec(memory_space=pltpu.SMEM),
      ],
      out_shape=[
          jax.ShapeDtypeStruct((N, _DP), jnp.bfloat16),
          jax.ShapeDtypeStruct((1, 1), jnp.float32),
      ],
  )(pred, maskf)

  out = pl.pallas_call(
      _loss_body,
      grid=(_NB + 1,),
      in_specs=[
          pl.BlockSpec((N, _DP), lambda j: (0, 0)),
          pl.BlockSpec((_BE, D), lambda j: (jnp.minimum(j, _NB - 1), 0)),
          pl.BlockSpec(memory_space=pltpu.SMEM),
      ],
      out_specs=pl.BlockSpec(memory_space=pltpu.SMEM),
      out_shape=jax.ShapeDtypeStruct((1, 1), jnp.float32),
      scratch_shapes=[pltpu.VMEM((N, _DP), jnp.bfloat16)],
  )(a_mat, emb, msum)
  return out[0, 0]


# speedup vs baseline: 1.0821x; 1.0697x over previous
import functools

import jax
import jax.numpy as jnp
from jax import lax
from jax.experimental import pallas as pl
from jax.experimental.pallas import tpu as pltpu
from jax.experimental.pallas import tpu_sc as plsc

N = 2048
D = 1024
K = 8192

_info = plsc.get_sparse_core_info()
_NC = _info.num_cores
_NS = _info.num_subcores
_NW = _NC * _NS
_BPW = N // _NW
_HPW = _BPW // 2


def _sc_gather(codebook, target):
  mesh = plsc.VectorSubcoreMesh(core_axis_name="c", subcore_axis_name="s")

  @functools.partial(
      pl.kernel,
      mesh=mesh,
      out_type=jax.ShapeDtypeStruct((N, D), jnp.float32),
      scratch_types=[
          pltpu.VMEM((_HPW,), jnp.int32),
          pltpu.VMEM((_HPW,), jnp.int32),
          pltpu.VMEM((_HPW, D), jnp.float32),
          pltpu.VMEM((_HPW, D), jnp.float32),
          pltpu.SemaphoreType.DMA,
          pltpu.SemaphoreType.DMA,
      ],
  )
  def k(table_hbm, idx_hbm, out_hbm, idx0, idx1, rows0, rows1, sem0, sem1):
    wid = lax.axis_index("s") * _NC + lax.axis_index("c")
    base = wid * _BPW
    pltpu.sync_copy(idx_hbm.at[pl.ds(base, _HPW)], idx0)
    pltpu.sync_copy(idx_hbm.at[pl.ds(base + _HPW, _HPW)], idx1)
    cp0 = pltpu.async_copy(table_hbm.at[idx0], rows0, sem0)
    cp0.wait()
    cp1 = pltpu.async_copy(table_hbm.at[idx1], rows1, sem1)
    pltpu.sync_copy(rows0, out_hbm.at[pl.ds(base, _HPW)])
    cp1.wait()
    pltpu.sync_copy(rows1, out_hbm.at[pl.ds(base + _HPW, _HPW)])

  return k(codebook, target)


_BJ = 256


def _loss_body(pred_ref, e_ref, mask_ref, out_ref, p2_ref):
  j = pl.program_id(0)
  nj = pl.num_programs(0)

  @pl.when(j == 0)
  def _():
    p2_ref[...] = jnp.sum(pred_ref[...] * pred_ref[...], axis=1,
                          keepdims=True)
    out_ref[0, 0] = 0.0

  e = e_ref[...]
  g = lax.dot_general(pred_ref[...], e, (((1,), (1,)), ((), ())),
                      preferred_element_type=jnp.float32)
  t2 = jnp.sum(e * e, axis=1)
  d2 = p2_ref[...] + t2[None, :] - 2.0 * g
  d2 = jnp.maximum(d2, 1e-30)
  part = jnp.sum((d2 * lax.rsqrt(d2)) * mask_ref[...])
  acc = out_ref[0, 0] + part

  @pl.when(j < nj - 1)
  def _():
    out_ref[0, 0] = acc

  @pl.when(j == nj - 1)
  def _():
    msum = jnp.sum(mask_ref[...])
    out_ref[0, 0] = acc / (msum * D)


def kernel(pred, target, codebook):
  emb = _sc_gather(codebook, target)
  maskf = (target != -1).astype(jnp.float32).reshape(N, 1)

  out = pl.pallas_call(
      _loss_body,
      grid=(N // _BJ,),
      in_specs=[
          pl.BlockSpec((N, D), lambda j: (0, 0)),
          pl.BlockSpec((_BJ, D), lambda j: (j, 0)),
          pl.BlockSpec((N, 1), lambda j: (0, 0)),
      ],
      out_specs=pl.BlockSpec(memory_space=pltpu.SMEM),
      out_shape=jax.ShapeDtypeStruct((1, 1), jnp.float32),
      scratch_shapes=[pltpu.VMEM((N, 1), jnp.float32)],
  )(pred, emb, maskf)
  return out[0, 0]
